# Initial kernel scaffold; baseline (speedup 1.0000x reference)
import jax
import jax.numpy as jnp
from jax.experimental import pallas as pl


def kernel(x, emb_table, W1, b1):
    raise NotImplementedError("write your pallas kernel here")


# speedup vs baseline: 23.4136x; 23.4136x over previous
import functools

import jax
import jax.numpy as jnp
from jax import lax
from jax.experimental import pallas as pl
from jax.experimental.pallas import tpu as pltpu
from jax.experimental.pallas import tpu_sc as plsc

_VOCAB = 100000
_HIDDEN = 128
_B = 4096
_L = 200
_TOT = _B * _L

_NC = 2
_NS = 16
_NW = _NC * _NS
_LANES = 16

_N_PER_W = _TOT // _NW
_CHUNK = 6400
_N_CHUNKS = _N_PER_W // _CHUNK


_ROWS_BLK = 4096


def _table_body(emb_ref, w_ref, b_ref, t_ref):
    acc = jnp.sum(emb_ref[...] * w_ref[...], axis=1) + b_ref[0]
    t_ref[...] = jax.nn.sigmoid(acc)


def _compute_table(emb_table, w_row, b1):
    return pl.pallas_call(
        _table_body,
        grid=(pl.cdiv(_VOCAB, _ROWS_BLK),),
        in_specs=[
            pl.BlockSpec((_ROWS_BLK, _HIDDEN), lambda i: (i, 0)),
            pl.BlockSpec((1, _HIDDEN), lambda i: (0, 0)),
            pl.BlockSpec(memory_space=pltpu.SMEM),
        ],
        out_specs=pl.BlockSpec((_ROWS_BLK,), lambda i: (i,)),
        out_shape=jax.ShapeDtypeStruct((_VOCAB,), jnp.float32),
    )(emb_table, w_row, b1)


@functools.lru_cache(maxsize=1)
def _build_gather_kernel():
    mesh = plsc.VectorSubcoreMesh(core_axis_name="c", subcore_axis_name="s")

    @functools.partial(
        pl.kernel,
        mesh=mesh,
        out_type=jax.ShapeDtypeStruct((_TOT,), jnp.float32),
        scratch_types=[
            pltpu.VMEM((_VOCAB,), jnp.float32),
            pltpu.VMEM((_CHUNK,), jnp.int32),
            pltpu.VMEM((_CHUNK,), jnp.float32),
        ],
        compiler_params=pltpu.CompilerParams(needs_layout_passes=False),
    )
    def _gather_kernel(t_hbm, idx_hbm, out_hbm, t_v, idx_v, out_v):
        wid = lax.axis_index("s") * _NC + lax.axis_index("c")
        base = wid * _N_PER_W
        pltpu.sync_copy(t_hbm, t_v)

        def chunk_body(ci, carry):
            off = base + ci * _CHUNK
            pltpu.sync_copy(idx_hbm.at[pl.ds(off, _CHUNK)], idx_v)

            def body(i, c):
                ids = idx_v[pl.ds(i * _LANES, _LANES)]
                out_v[pl.ds(i * _LANES, _LANES)] = plsc.load_gather(t_v, [ids])
                return c

            lax.fori_loop(0, _CHUNK // _LANES, body, 0, unroll=4)
            pltpu.sync_copy(out_v, out_hbm.at[pl.ds(off, _CHUNK)])
            return carry

        lax.fori_loop(0, _N_CHUNKS, chunk_body, 0)

    return _gather_kernel


def kernel(x, emb_table, W1, b1):
    w_row = W1.reshape(1, _HIDDEN).astype(jnp.float32)
    t = _compute_table(emb_table, w_row, b1.astype(jnp.float32))
    idx = x.reshape(_TOT).astype(jnp.int32)
    out = _build_gather_kernel()(t, idx)
    return out.reshape(_B, _L, 1)


# speedup vs baseline: 34.5520x; 1.4757x over previous
import functools

import jax
import jax.numpy as jnp
from jax import lax
from jax.experimental import pallas as pl
from jax.experimental.pallas import tpu as pltpu
from jax.experimental.pallas import tpu_sc as plsc

_VOCAB = 100000
_HIDDEN = 128
_B = 4096
_L = 200
_TOT = _B * _L

_NC = 2
_NS = 16
_NW = _NC * _NS
_LANES = 16

_N_PER_W = _TOT // _NW
_CHUNK = 6400
_N_CHUNKS = _N_PER_W // _CHUNK


_ROWS_BLK = 4096


def _table_body(emb_ref, w_ref, b_ref, t_ref):
    rows = []
    for s in range(_ROWS_BLK // 128):
        es = emb_ref[pl.ds(s * 128, 128), :]
        rows.append(jax.lax.dot_general(
            w_ref[...], es, (((1,), (1,)), ((), ())),
            preferred_element_type=jnp.float32))
    tile = jnp.concatenate(rows, axis=0) + b_ref[0]
    t_ref[...] = jax.nn.sigmoid(tile).reshape(_ROWS_BLK)


def _compute_table(emb_table, w_row, b1):
    return pl.pallas_call(
        _table_body,
        grid=(pl.cdiv(_VOCAB, _ROWS_BLK),),
        in_specs=[
            pl.BlockSpec((_ROWS_BLK, _HIDDEN), lambda i: (i, 0)),
            pl.BlockSpec((1, _HIDDEN), lambda i: (0, 0)),
            pl.BlockSpec(memory_space=pltpu.SMEM),
        ],
        out_specs=pl.BlockSpec((_ROWS_BLK,), lambda i: (i,)),
        out_shape=jax.ShapeDtypeStruct((_VOCAB,), jnp.float32),
    )(emb_table, w_row, b1)


@functools.lru_cache(maxsize=1)
def _build_gather_kernel():
    mesh = plsc.VectorSubcoreMesh(core_axis_name="c", subcore_axis_name="s")

    @functools.partial(
        pl.kernel,
        mesh=mesh,
        out_type=jax.ShapeDtypeStruct((_TOT,), jnp.float32),
        scratch_types=[
            pltpu.VMEM((_VOCAB,), jnp.float32),
            pltpu.VMEM((_CHUNK,), jnp.int32),
            pltpu.VMEM((_CHUNK,), jnp.int32),
            pltpu.VMEM((_CHUNK,), jnp.float32),
            pltpu.VMEM((_CHUNK,), jnp.float32),
            pltpu.SemaphoreType.DMA,
            pltpu.SemaphoreType.DMA,
            pltpu.SemaphoreType.DMA,
            pltpu.SemaphoreType.DMA,
            pltpu.SemaphoreType.DMA,
        ],
        compiler_params=pltpu.CompilerParams(needs_layout_passes=False),
    )
    def _gather_kernel(t_hbm, idx_hbm, out_hbm,
                       t_v, idx_a, idx_b, out_a, out_b,
                       sem_t, sem_ia, sem_ib, sem_oa, sem_ob):
        wid = lax.axis_index("s") * _NC + lax.axis_index("c")
        base = wid * _N_PER_W
        idx_bufs = (idx_a, idx_b)
        out_bufs = (out_a, out_b)
        in_sems = (sem_ia, sem_ib)
        out_sems = (sem_oa, sem_ob)

        cp_t = pltpu.async_copy(t_hbm, t_v, sem_t)
        in_cps = [pltpu.async_copy(
            idx_hbm.at[pl.ds(base, _CHUNK)], idx_a, sem_ia)]
        out_cps = [None, None]

        for ci in range(_N_CHUNKS):
            b = ci % 2
            if ci + 1 < _N_CHUNKS:
                in_cps.append(pltpu.async_copy(
                    idx_hbm.at[pl.ds(base + (ci + 1) * _CHUNK, _CHUNK)],
                    idx_bufs[(ci + 1) % 2], in_sems[(ci + 1) % 2]))
            in_cps[ci].wait()
            if ci == 0:
                cp_t.wait()
            if out_cps[b] is not None:
                out_cps[b].wait()
            idx_v = idx_bufs[b]
            out_v = out_bufs[b]

            @plsc.parallel_loop(0, _CHUNK, _LANES, unroll=8)
            def _gather_body(i):
                ids = idx_v[pl.ds(i, _LANES)]
                out_v[pl.ds(i, _LANES)] = plsc.load_gather(t_v, [ids])

            out_cps[b] = pltpu.async_copy(
                out_v, out_hbm.at[pl.ds(base + ci * _CHUNK, _CHUNK)],
                out_sems[b])

        for cp in out_cps:
            if cp is not None:
                cp.wait()

    return _gather_kernel


def kernel(x, emb_table, W1, b1):
    w_row = W1.reshape(1, _HIDDEN).astype(jnp.float32)
    t = _compute_table(emb_table, w_row, b1.astype(jnp.float32))
    idx = x.reshape(_TOT).astype(jnp.int32)
    out = _build_gather_kernel()(t, idx)
    return out.reshape(_B, _L, 1)


# speedup vs baseline: 37.8766x; 1.0962x over previous
import functools

import jax
import jax.numpy as jnp
from jax import lax
from jax.experimental import pallas as pl
from jax.experimental.pallas import tpu as pltpu
from jax.experimental.pallas import tpu_sc as plsc

_VOCAB = 100000
_HIDDEN = 128
_B = 4096
_L = 200
_TOT = _B * _L

_NC = 2
_NS = 16
_NW = _NC * _NS
_LANES = 16

_N_PER_W = _TOT // _NW
_CHUNK = 6400
_N_CHUNKS = _N_PER_W // _CHUNK


_ROWS_BLK = 16384


def _table_body(emb_ref, w_ref, b_ref, t_ref):
    rows = []
    for s in range(_ROWS_BLK // 128):
        es = emb_ref[pl.ds(s * 128, 128), :]
        rows.append(jax.lax.dot_general(
            w_ref[...], es, (((1,), (1,)), ((), ())),
            preferred_element_type=jnp.float32))
    tile = jnp.concatenate(rows, axis=0) + b_ref[0]
    t_ref[...] = jax.nn.sigmoid(tile).reshape(_ROWS_BLK)


def _compute_table(emb_table, w_row, b1):
    return pl.pallas_call(
        _table_body,
        grid=(pl.cdiv(_VOCAB, _ROWS_BLK),),
        in_specs=[
            pl.BlockSpec((_ROWS_BLK, _HIDDEN), lambda i: (i, 0)),
            pl.BlockSpec((1, _HIDDEN), lambda i: (0, 0)),
            pl.BlockSpec(memory_space=pltpu.SMEM),
        ],
        out_specs=pl.BlockSpec((_ROWS_BLK,), lambda i: (i,)),
        out_shape=jax.ShapeDtypeStruct((_VOCAB,), jnp.float32),
    )(emb_table, w_row, b1)


@functools.lru_cache(maxsize=1)
def _build_gather_kernel():
    mesh = plsc.VectorSubcoreMesh(core_axis_name="c", subcore_axis_name="s")

    @functools.partial(
        pl.kernel,
        mesh=mesh,
        out_type=jax.ShapeDtypeStruct((_TOT,), jnp.float32),
        scratch_types=[
            pltpu.VMEM((_VOCAB,), jnp.float32),
            pltpu.VMEM((_CHUNK,), jnp.int32),
            pltpu.VMEM((_CHUNK,), jnp.int32),
            pltpu.VMEM((_CHUNK,), jnp.float32),
            pltpu.VMEM((_CHUNK,), jnp.float32),
            pltpu.SemaphoreType.DMA,
            pltpu.SemaphoreType.DMA,
            pltpu.SemaphoreType.DMA,
            pltpu.SemaphoreType.DMA,
            pltpu.SemaphoreType.DMA,
        ],
        compiler_params=pltpu.CompilerParams(needs_layout_passes=False),
    )
    def _gather_kernel(t_hbm, idx_hbm, out_hbm,
                       t_v, idx_a, idx_b, out_a, out_b,
                       sem_t, sem_ia, sem_ib, sem_oa, sem_ob):
        wid = lax.axis_index("s") * _NC + lax.axis_index("c")
        base = wid * _N_PER_W
        idx_bufs = (idx_a, idx_b)
        out_bufs = (out_a, out_b)
        in_sems = (sem_ia, sem_ib)
        out_sems = (sem_oa, sem_ob)

        cp_t = pltpu.async_copy(t_hbm, t_v, sem_t)
        in_cps = [pltpu.async_copy(
            idx_hbm.at[pl.ds(base, _CHUNK)], idx_a, sem_ia)]
        out_cps = [None, None]

        for ci in range(_N_CHUNKS):
            b = ci % 2
            if ci + 1 < _N_CHUNKS:
                in_cps.append(pltpu.async_copy(
                    idx_hbm.at[pl.ds(base + (ci + 1) * _CHUNK, _CHUNK)],
                    idx_bufs[(ci + 1) % 2], in_sems[(ci + 1) % 2]))
            in_cps[ci].wait()
            if ci == 0:
                cp_t.wait()
            if out_cps[b] is not None:
                out_cps[b].wait()
            idx_v = idx_bufs[b]
            out_v = out_bufs[b]

            @plsc.parallel_loop(0, _CHUNK, _LANES, unroll=8)
            def _gather_body(i):
                ids = idx_v[pl.ds(i, _LANES)]
                out_v[pl.ds(i, _LANES)] = plsc.load_gather(t_v, [ids])

            out_cps[b] = pltpu.async_copy(
                out_v, out_hbm.at[pl.ds(base + ci * _CHUNK, _CHUNK)],
                out_sems[b])

        for cp in out_cps:
            if cp is not None:
                cp.wait()

    return _gather_kernel


def kernel(x, emb_table, W1, b1):
    w_row = W1.reshape(1, _HIDDEN).astype(jnp.float32)
    t = _compute_table(emb_table, w_row, b1.astype(jnp.float32))
    idx = x.reshape(_TOT).astype(jnp.int32)
    out = _build_gather_kernel()(t, idx)
    return out.reshape(_B, _L, 1)


# speedup vs baseline: 40.8159x; 1.0776x over previous
import functools

import jax
import jax.numpy as jnp
from jax import lax
from jax.experimental import pallas as pl
from jax.experimental.pallas import tpu as pltpu
from jax.experimental.pallas import tpu_sc as plsc

_VOCAB = 100000
_HIDDEN = 128
_B = 4096
_L = 200
_TOT = _B * _L

_NC = 2
_NS = 16
_NW = _NC * _NS
_LANES = 16

_N_PER_W = _TOT // _NW
_CHUNK = 6400
_N_CHUNKS = _N_PER_W // _CHUNK


_SPLIT = 51200
_ROWS_BLK = 10240


def _round_bf16_bits(v):
    u = jax.lax.bitcast_convert_type(v, jnp.int32)
    return (u + 0x7FFF + ((u >> 16) & 1)) >> 16


def _table_body(emb_lo_ref, emb_hi_ref, w_ref, b_ref, t_ref):
    def half(ref):
        rows = []
        for s in range(_ROWS_BLK // 128):
            es = ref[pl.ds(s * 128, 128), :]
            rows.append(jax.lax.dot_general(
                w_ref[...], es, (((1,), (1,)), ((), ())),
                preferred_element_type=jnp.float32))
        return jax.nn.sigmoid(jnp.concatenate(rows, axis=0) + b_ref[0])

    lo = _round_bf16_bits(half(emb_lo_ref))
    hi = _round_bf16_bits(half(emb_hi_ref))
    t_ref[...] = (lo | (hi << 16)).reshape(_ROWS_BLK)


def _compute_table(emb_table, w_row, b1):
    n_hi_blk = _SPLIT // _ROWS_BLK
    return pl.pallas_call(
        _table_body,
        grid=(_SPLIT // _ROWS_BLK,),
        in_specs=[
            pl.BlockSpec((_ROWS_BLK, _HIDDEN), lambda i: (i, 0)),
            pl.BlockSpec((_ROWS_BLK, _HIDDEN), lambda i: (i + n_hi_blk, 0)),
            pl.BlockSpec((1, _HIDDEN), lambda i: (0, 0)),
            pl.BlockSpec(memory_space=pltpu.SMEM),
        ],
        out_specs=pl.BlockSpec((_ROWS_BLK,), lambda i: (i,)),
        out_shape=jax.ShapeDtypeStruct((_SPLIT,), jnp.int32),
    )(emb_table, emb_table, w_row, b1)


@functools.lru_cache(maxsize=1)
def _build_gather_kernel():
    mesh = plsc.VectorSubcoreMesh(core_axis_name="c", subcore_axis_name="s")

    @functools.partial(
        pl.kernel,
        mesh=mesh,
        out_type=jax.ShapeDtypeStruct((_TOT,), jnp.float32),
        scratch_types=[
            pltpu.VMEM((_SPLIT,), jnp.int32),
            pltpu.VMEM((_CHUNK,), jnp.int32),
            pltpu.VMEM((_CHUNK,), jnp.int32),
            pltpu.VMEM((_CHUNK,), jnp.float32),
            pltpu.VMEM((_CHUNK,), jnp.float32),
            pltpu.SemaphoreType.DMA,
            pltpu.SemaphoreType.DMA,
            pltpu.SemaphoreType.DMA,
            pltpu.SemaphoreType.DMA,
            pltpu.SemaphoreType.DMA,
        ],
        compiler_params=pltpu.CompilerParams(needs_layout_passes=False),
    )
    def _gather_kernel(t_hbm, idx_hbm, out_hbm,
                       t_v, idx_a, idx_b, out_a, out_b,
                       sem_t, sem_ia, sem_ib, sem_oa, sem_ob):
        wid = lax.axis_index("s") * _NC + lax.axis_index("c")
        base = wid * _N_PER_W
        idx_bufs = (idx_a, idx_b)
        out_bufs = (out_a, out_b)
        in_sems = (sem_ia, sem_ib)
        out_sems = (sem_oa, sem_ob)

        cp_t = pltpu.async_copy(t_hbm, t_v, sem_t)
        in_cps = [pltpu.async_copy(
            idx_hbm.at[pl.ds(base, _CHUNK)], idx_a, sem_ia)]
        out_cps = [None, None]

        for ci in range(_N_CHUNKS):
            b = ci % 2
            if ci + 1 < _N_CHUNKS:
                in_cps.append(pltpu.async_copy(
                    idx_hbm.at[pl.ds(base + (ci + 1) * _CHUNK, _CHUNK)],
                    idx_bufs[(ci + 1) % 2], in_sems[(ci + 1) % 2]))
            in_cps[ci].wait()
            if ci == 0:
                cp_t.wait()
            if out_cps[b] is not None:
                out_cps[b].wait()
            idx_v = idx_bufs[b]
            out_v = out_bufs[b]

            @plsc.parallel_loop(0, _CHUNK, _LANES, unroll=8)
            def _gather_body(i):
                ids = idx_v[pl.ds(i, _LANES)]
                is_hi = ids >= _SPLIT
                widx = ids - jnp.where(is_hi, _SPLIT, 0)
                word = plsc.load_gather(t_v, [widx])
                half = jax.lax.shift_right_logical(
                    word, jnp.where(is_hi, 16, 0))
                bits = (half & 0xFFFF) << 16
                out_v[pl.ds(i, _LANES)] = jax.lax.bitcast_convert_type(
                    bits, jnp.float32)

            out_cps[b] = pltpu.async_copy(
                out_v, out_hbm.at[pl.ds(base + ci * _CHUNK, _CHUNK)],
                out_sems[b])

        for cp in out_cps:
            if cp is not None:
                cp.wait()

    return _gather_kernel


def kernel(x, emb_table, W1, b1):
    w_row = W1.reshape(1, _HIDDEN).astype(jnp.float32)
    t = _compute_table(emb_table, w_row, b1.astype(jnp.float32))
    idx = x.reshape(_TOT).astype(jnp.int32)
    out = _build_gather_kernel()(t, idx)
    return out.reshape(_B, _L, 1)
